# Initial kernel scaffold; baseline (speedup 1.0000x reference)
import jax
import jax.numpy as jnp
from jax.experimental import pallas as pl


def kernel(x, edge_index, W1_self, W1_neigh, b1, W2_self, W2_neigh, b2):
    raise NotImplementedError("write your pallas kernel here")


# speedup vs baseline: 2.7629x; 2.7629x over previous
import functools

import jax
import jax.numpy as jnp
from jax import lax
from jax.experimental import pallas as pl
from jax.experimental.pallas import tpu as pltpu
from jax.experimental.pallas import tpu_sc as plsc

N_NODES = 10000
D = 128
NC = 2
NS = 16
NW = NC * NS
CHUNK = 128
E_PAD = 327680
E_PER_W = E_PAD // NW
N_CHUNKS = E_PER_W // CHUNK
N_PAD = 10240
ROWS_PER_TILE = N_PAD // NS
STAGE = 32
DEG_W = 16


def _make_agg():
  mesh = plsc.VectorSubcoreMesh(core_axis_name="c", subcore_axis_name="s")
  scratch = [
      pltpu.VMEM_SHARED((N_PAD, D), jnp.float32),
      pltpu.VMEM((CHUNK,), jnp.int32),
      pltpu.VMEM((CHUNK,), jnp.int32),
      pltpu.VMEM((CHUNK, D), jnp.float32),
      pltpu.VMEM((STAGE, D), jnp.float32),
      pltpu.SemaphoreType.DMA,
  ]

  def body(feats, src, dst, zrows, acc_out, acc, src_v, dst_v, rows, stage,
           sem):
    cid = lax.axis_index("c")
    sid = lax.axis_index("s")
    wid = sid * NC + cid
    r0 = sid * ROWS_PER_TILE

    pltpu.sync_copy(zrows, stage)

    def zc(i, _):
      pltpu.sync_copy(stage, acc.at[pl.ds(r0 + i * STAGE, STAGE)])
      return 0
    lax.fori_loop(0, ROWS_PER_TILE // STAGE, zc, 0)

    plsc.subcore_barrier()

    base = wid * E_PER_W

    def step(c, _):
      off = base + c * CHUNK
      pltpu.sync_copy(src.at[pl.ds(off, CHUNK)], src_v)
      pltpu.sync_copy(dst.at[pl.ds(off, CHUNK)], dst_v)
      pltpu.async_copy(feats.at[src_v], rows, sem).wait()
      pltpu.sync_copy(rows, acc.at[dst_v], add=True)
      return 0
    lax.fori_loop(0, N_CHUNKS, step, 0)

    plsc.subcore_barrier()

    def oc(i, _):
      pltpu.sync_copy(acc.at[pl.ds(r0 + i * STAGE, STAGE)], stage)
      pltpu.sync_copy(stage, acc_out.at[cid, pl.ds(r0 + i * STAGE, STAGE)])
      return 0
    lax.fori_loop(0, ROWS_PER_TILE // STAGE, oc, 0)

  return pl.kernel(
      body,
      out_type=jax.ShapeDtypeStruct((NC, N_PAD, D), jnp.float32),
      mesh=mesh,
      scratch_types=scratch,
  )


def _make_deg():
  mesh = plsc.VectorSubcoreMesh(core_axis_name="c", subcore_axis_name="s")
  scratch = [
      pltpu.VMEM_SHARED((N_PAD, D), jnp.float32),
      pltpu.VMEM((CHUNK,), jnp.int32),
      pltpu.VMEM((CHUNK, D), jnp.float32),
      pltpu.VMEM((STAGE, D), jnp.float32),
      pltpu.SemaphoreType.DMA,
  ]

  def body(dst, zrows, ones, deg_out, dacc, dst_v, ones_v, dstage, sem):
    cid = lax.axis_index("c")
    sid = lax.axis_index("s")
    wid = sid * NC + cid
    r0 = sid * ROWS_PER_TILE

    pltpu.sync_copy(zrows, dstage)

    def zd(i, _):
      pltpu.sync_copy(dstage, dacc.at[pl.ds(r0 + i * STAGE, STAGE)])
      return 0
    lax.fori_loop(0, ROWS_PER_TILE // STAGE, zd, 0)

    pltpu.sync_copy(ones, ones_v)

    plsc.subcore_barrier()

    base = wid * E_PER_W

    def step(c, _):
      off = base + c * CHUNK
      pltpu.sync_copy(dst.at[pl.ds(off, CHUNK)], dst_v)
      pltpu.sync_copy(ones_v, dacc.at[dst_v], add=True)
      return 0
    lax.fori_loop(0, N_CHUNKS, step, 0)

    plsc.subcore_barrier()

    def od(i, _):
      pltpu.sync_copy(dacc.at[pl.ds(r0 + i * STAGE, STAGE)], dstage)
      pltpu.sync_copy(dstage, deg_out.at[cid, pl.ds(r0 + i * STAGE, STAGE)])
      return 0
    lax.fori_loop(0, ROWS_PER_TILE // STAGE, od, 0)

  return pl.kernel(
      body,
      out_type=jax.ShapeDtypeStruct((NC, N_PAD, D), jnp.float32),
      mesh=mesh,
      scratch_types=scratch,
  )


_agg = _make_agg()
_deg = _make_deg()

_TC_ROWS = 2000


def _make_combine(relu: bool):
  def body(x_ref, sa_ref, sb_ref, da_ref, db_ref, ws_ref, wn_ref, b_ref, o_ref):
    deg = jnp.maximum(da_ref[:, 0:1] + db_ref[:, 0:1], 1.0)
    hn = (sa_ref[...] + sb_ref[...]) / deg
    out = (
        jnp.dot(x_ref[...], ws_ref[...], preferred_element_type=jnp.float32,
                precision=lax.Precision.HIGHEST)
        + jnp.dot(hn, wn_ref[...], preferred_element_type=jnp.float32,
                  precision=lax.Precision.HIGHEST)
        + b_ref[...]
    )
    if relu:
      out = jnp.maximum(out, 0.0)
    o_ref[...] = out

  return pl.pallas_call(
      body,
      grid=(N_NODES // _TC_ROWS,),
      in_specs=[
          pl.BlockSpec((_TC_ROWS, D), lambda i: (i, 0)),
          pl.BlockSpec((_TC_ROWS, D), lambda i: (i, 0)),
          pl.BlockSpec((_TC_ROWS, D), lambda i: (i, 0)),
          pl.BlockSpec((_TC_ROWS, D), lambda i: (i, 0)),
          pl.BlockSpec((_TC_ROWS, D), lambda i: (i, 0)),
          pl.BlockSpec((D, D), lambda i: (0, 0)),
          pl.BlockSpec((D, D), lambda i: (0, 0)),
          pl.BlockSpec((1, D), lambda i: (0, 0)),
      ],
      out_specs=pl.BlockSpec((_TC_ROWS, D), lambda i: (i, 0)),
      out_shape=jax.ShapeDtypeStruct((N_NODES, D), jnp.float32),
  )


_combine_relu = _make_combine(True)
_combine_lin = _make_combine(False)


def kernel(x, edge_index, W1_self, W1_neigh, b1, W2_self, W2_neigh, b2):
  src = edge_index[0].astype(jnp.int32)
  dst = edge_index[1].astype(jnp.int32)
  n_edges = src.shape[0]
  pad = E_PAD - n_edges
  src_p = jnp.concatenate([src, jnp.zeros((pad,), jnp.int32)])
  dst_p = jnp.concatenate([dst, jnp.full((pad,), N_NODES, jnp.int32)])

  zrows = jnp.zeros((STAGE, D), jnp.float32)
  ones = jnp.ones((CHUNK, D), jnp.float32)
  degs = _deg(dst_p, zrows, ones)
  acc1 = _agg(x, src_p, dst_p, zrows)
  da = degs[0, :N_NODES]
  db = degs[1, :N_NODES]
  h = _combine_relu(x, acc1[0, :N_NODES], acc1[1, :N_NODES], da, db,
                    W1_self, W1_neigh, b1.reshape(1, D))
  acc2 = _agg(h, src_p, dst_p, zrows)
  out = _combine_lin(h, acc2[0, :N_NODES], acc2[1, :N_NODES], da, db,
                     W2_self, W2_neigh, b2.reshape(1, D))
  return out


# speedup vs baseline: 3.2579x; 1.1791x over previous
import functools

import jax
import jax.numpy as jnp
from jax import lax
from jax.experimental import pallas as pl
from jax.experimental.pallas import tpu as pltpu
from jax.experimental.pallas import tpu_sc as plsc

N_NODES = 10000
D = 128
NC = 2
NS = 16
NW = NC * NS
CHUNK = 96
N_CHUNKS = 106
E_PER_W = N_CHUNKS * CHUNK
E_PAD = NW * E_PER_W
N_PAD = 10240
ROWS_PER_TILE = N_PAD // NS
STAGE = 32
DEG_W = 16


def _make_agg():
  mesh = plsc.VectorSubcoreMesh(core_axis_name="c", subcore_axis_name="s")
  scratch = [
      pltpu.VMEM_SHARED((N_PAD, D), jnp.float32),
      pltpu.VMEM((CHUNK,), jnp.int32),
      pltpu.VMEM((CHUNK,), jnp.int32),
      pltpu.VMEM((CHUNK,), jnp.int32),
      pltpu.VMEM((CHUNK,), jnp.int32),
      pltpu.VMEM((CHUNK, D), jnp.float32),
      pltpu.VMEM((CHUNK, D), jnp.float32),
      pltpu.VMEM((STAGE, D), jnp.float32),
      pltpu.SemaphoreType.DMA,
      pltpu.SemaphoreType.DMA,
      pltpu.SemaphoreType.DMA,
      pltpu.SemaphoreType.DMA,
  ]

  def body(feats, src, dst, zrows, acc_out, acc, src0, dst0, src1, dst1,
           rows0, rows1, stage, gsem0, gsem1, ssem0, ssem1):
    cid = lax.axis_index("c")
    sid = lax.axis_index("s")
    wid = sid * NC + cid
    r0 = sid * ROWS_PER_TILE

    pltpu.sync_copy(zrows, stage)

    def zc(i, _):
      pltpu.sync_copy(stage, acc.at[pl.ds(r0 + i * STAGE, STAGE)])
      return 0
    lax.fori_loop(0, ROWS_PER_TILE // STAGE, zc, 0)

    plsc.subcore_barrier()

    base = wid * E_PER_W
    slots = ((src0, dst0, rows0, gsem0, ssem0),
             (src1, dst1, rows1, gsem1, ssem1))

    pltpu.sync_copy(src.at[pl.ds(base, CHUNK)], src0)
    pltpu.sync_copy(dst.at[pl.ds(base, CHUNK)], dst0)
    pltpu.async_copy(feats.at[src0], rows0, gsem0)

    def chunk_step(c, k):
      src_s, dst_s, rows_s, gsem_s, ssem_s = slots[k]
      src_t, dst_t, rows_t, gsem_t, ssem_t = slots[1 - k]
      pltpu.make_async_copy(feats.at[src_s], rows_s, gsem_s).wait()
      @pl.when(c > 0)
      def _():
        pltpu.make_async_copy(rows_t, acc.at[dst_t], ssem_t).wait()
      cn = jnp.minimum(c + 1, N_CHUNKS - 1)
      offn = base + cn * CHUNK
      pltpu.sync_copy(src.at[pl.ds(offn, CHUNK)], src_t)
      pltpu.sync_copy(dst.at[pl.ds(offn, CHUNK)], dst_t)
      pltpu.async_copy(feats.at[src_t], rows_t, gsem_t)
      pltpu.async_copy(rows_s, acc.at[dst_s], ssem_s, add=True)

    def pair(g, _):
      chunk_step(2 * g, 0)
      chunk_step(2 * g + 1, 1)
      return 0
    lax.fori_loop(0, N_CHUNKS // 2, pair, 0)

    pltpu.make_async_copy(feats.at[src0], rows0, gsem0).wait()
    pltpu.make_async_copy(rows1, acc.at[dst1], ssem1).wait()

    plsc.subcore_barrier()

    def oc(i, _):
      pltpu.sync_copy(acc.at[pl.ds(r0 + i * STAGE, STAGE)], stage)
      pltpu.sync_copy(stage, acc_out.at[cid, pl.ds(r0 + i * STAGE, STAGE)])
      return 0
    lax.fori_loop(0, ROWS_PER_TILE // STAGE, oc, 0)

  return pl.kernel(
      body,
      out_type=jax.ShapeDtypeStruct((NC, N_PAD, D), jnp.float32),
      mesh=mesh,
      scratch_types=scratch,
  )


def _make_deg():
  mesh = plsc.VectorSubcoreMesh(core_axis_name="c", subcore_axis_name="s")
  scratch = [
      pltpu.VMEM_SHARED((N_PAD, D), jnp.float32),
      pltpu.VMEM((CHUNK,), jnp.int32),
      pltpu.VMEM((CHUNK, D), jnp.float32),
      pltpu.VMEM((STAGE, D), jnp.float32),
      pltpu.SemaphoreType.DMA,
  ]

  def body(dst, zrows, ones, deg_out, dacc, dst_v, ones_v, dstage, sem):
    cid = lax.axis_index("c")
    sid = lax.axis_index("s")
    wid = sid * NC + cid
    r0 = sid * ROWS_PER_TILE

    pltpu.sync_copy(zrows, dstage)

    def zd(i, _):
      pltpu.sync_copy(dstage, dacc.at[pl.ds(r0 + i * STAGE, STAGE)])
      return 0
    lax.fori_loop(0, ROWS_PER_TILE // STAGE, zd, 0)

    pltpu.sync_copy(ones, ones_v)

    plsc.subcore_barrier()

    base = wid * E_PER_W

    def step(c, _):
      off = base + c * CHUNK
      pltpu.sync_copy(dst.at[pl.ds(off, CHUNK)], dst_v)
      pltpu.sync_copy(ones_v, dacc.at[dst_v], add=True)
      return 0
    lax.fori_loop(0, N_CHUNKS, step, 0)

    plsc.subcore_barrier()

    def od(i, _):
      pltpu.sync_copy(dacc.at[pl.ds(r0 + i * STAGE, STAGE)], dstage)
      pltpu.sync_copy(dstage, deg_out.at[cid, pl.ds(r0 + i * STAGE, STAGE)])
      return 0
    lax.fori_loop(0, ROWS_PER_TILE // STAGE, od, 0)

  return pl.kernel(
      body,
      out_type=jax.ShapeDtypeStruct((NC, N_PAD, D), jnp.float32),
      mesh=mesh,
      scratch_types=scratch,
  )


_agg = _make_agg()
_deg = _make_deg()

_TC_ROWS = 2000


def _make_combine(relu: bool):
  def body(x_ref, sa_ref, sb_ref, da_ref, db_ref, ws_ref, wn_ref, b_ref, o_ref):
    deg = jnp.maximum(da_ref[:, 0:1] + db_ref[:, 0:1], 1.0)
    hn = (sa_ref[...] + sb_ref[...]) / deg
    out = (
        jnp.dot(x_ref[...], ws_ref[...], preferred_element_type=jnp.float32,
                precision=lax.Precision.HIGHEST)
        + jnp.dot(hn, wn_ref[...], preferred_element_type=jnp.float32,
                  precision=lax.Precision.HIGHEST)
        + b_ref[...]
    )
    if relu:
      out = jnp.maximum(out, 0.0)
    o_ref[...] = out

  return pl.pallas_call(
      body,
      grid=(N_NODES // _TC_ROWS,),
      in_specs=[
          pl.BlockSpec((_TC_ROWS, D), lambda i: (i, 0)),
          pl.BlockSpec((_TC_ROWS, D), lambda i: (i, 0)),
          pl.BlockSpec((_TC_ROWS, D), lambda i: (i, 0)),
          pl.BlockSpec((_TC_ROWS, D), lambda i: (i, 0)),
          pl.BlockSpec((_TC_ROWS, D), lambda i: (i, 0)),
          pl.BlockSpec((D, D), lambda i: (0, 0)),
          pl.BlockSpec((D, D), lambda i: (0, 0)),
          pl.BlockSpec((1, D), lambda i: (0, 0)),
      ],
      out_specs=pl.BlockSpec((_TC_ROWS, D), lambda i: (i, 0)),
      out_shape=jax.ShapeDtypeStruct((N_NODES, D), jnp.float32),
  )


_combine_relu = _make_combine(True)
_combine_lin = _make_combine(False)


def kernel(x, edge_index, W1_self, W1_neigh, b1, W2_self, W2_neigh, b2):
  src = edge_index[0].astype(jnp.int32)
  dst = edge_index[1].astype(jnp.int32)
  n_edges = src.shape[0]
  pad = E_PAD - n_edges
  src_p = jnp.concatenate([src, jnp.zeros((pad,), jnp.int32)])
  dst_p = jnp.concatenate([dst, jnp.full((pad,), N_NODES, jnp.int32)])

  zrows = jnp.zeros((STAGE, D), jnp.float32)
  ones = jnp.ones((CHUNK, D), jnp.float32)
  degs = _deg(dst_p, zrows, ones)
  acc1 = _agg(x, src_p, dst_p, zrows)
  da = degs[0, :N_NODES]
  db = degs[1, :N_NODES]
  h = _combine_relu(x, acc1[0, :N_NODES], acc1[1, :N_NODES], da, db,
                    W1_self, W1_neigh, b1.reshape(1, D))
  acc2 = _agg(h, src_p, dst_p, zrows)
  out = _combine_lin(h, acc2[0, :N_NODES], acc2[1, :N_NODES], da, db,
                     W2_self, W2_neigh, b2.reshape(1, D))
  return out
